# Initial kernel scaffold; baseline (speedup 1.0000x reference)
import jax
import jax.numpy as jnp
from jax.experimental import pallas as pl


def kernel(token_fields, float_fields, token_seq_field, token_table, float_table, seq_table):
    raise NotImplementedError("write your pallas kernel here")


# speedup vs baseline: 3.0771x; 3.0771x over previous
import functools

import jax
import jax.numpy as jnp
from jax import lax
from jax.experimental import pallas as pl
from jax.experimental.pallas import tpu as pltpu
from jax.experimental.pallas import tpu_sc as plsc

B = 4096
N_TOK = 26
TOK_DIM = 10000
N_FLOAT = 13
SEQ_VOCAB = 100000
HIST = 50
HISTP = 64
D = 64

NC = 2
NSUB = 16
NW = NC * NSUB
PB = B // NW
NB = 4
NSTEPS = PB // NB
FFP = 16

_mesh = plsc.VectorSubcoreMesh(core_axis_name="c", subcore_axis_name="s")


@functools.partial(
    pl.kernel,
    out_type=(
        jax.ShapeDtypeStruct((B * (N_TOK + 1), D), jnp.float32),
        jax.ShapeDtypeStruct((B * N_FLOAT, D), jnp.float32),
    ),
    mesh=_mesh,
    scratch_types=(
        pltpu.VMEM((PB * N_TOK,), jnp.int32),
        pltpu.VMEM((PB * HISTP,), jnp.int32),
        pltpu.VMEM((PB * FFP,), jnp.float32),
        pltpu.VMEM((N_FLOAT * D,), jnp.float32),
        pltpu.VMEM((1, D), jnp.float32),
        pltpu.VMEM((NB * N_TOK, D), jnp.float32),
        pltpu.VMEM((NB * HIST, D), jnp.float32),
        pltpu.VMEM((NB, D), jnp.float32),
        pltpu.VMEM((NB * N_FLOAT, D), jnp.float32),
        pltpu.SemaphoreType.DMA,
        pltpu.SemaphoreType.DMA,
    ),
    compiler_params=pltpu.CompilerParams(use_tc_tiling_on_sc=False,
                                         needs_layout_passes=False),
)
def _sc_embed(tok_idx, seq_idx, ffp, ftab, tok_table, seq_table,
              out_sp, out_dn,
              idx_tok_v, idx_seq_v, ff_v, ftab_v, row0_v,
              tok_rows, seq_rows, pooled_v, dn_stage,
              sem_in, sem_out):
    wid = lax.axis_index("s") * NC + lax.axis_index("c")

    pltpu.sync_copy(tok_idx.at[pl.ds(wid * (PB * N_TOK), PB * N_TOK)], idx_tok_v)
    pltpu.sync_copy(seq_idx.at[pl.ds(wid * (PB * HISTP), PB * HISTP)], idx_seq_v)
    pltpu.sync_copy(ffp.at[pl.ds(wid * (PB * FFP), PB * FFP)], ff_v)
    pltpu.sync_copy(ftab, ftab_v)
    pltpu.sync_copy(seq_table.at[pl.ds(0, 1)], row0_v)

    lane = lax.iota(jnp.int32, 16)
    tail_mask = lane < 2

    def step(s, _):
        g0 = s * NB
        b0 = wid * PB + g0

        ct = pltpu.async_copy(
            tok_table.at[idx_tok_v.at[pl.ds(g0 * N_TOK, NB * N_TOK)]],
            tok_rows, sem_in)
        cseq = [
            pltpu.async_copy(
                seq_table.at[idx_seq_v.at[pl.ds((g0 + j) * HISTP, HIST)]],
                seq_rows.at[pl.ds(j * HIST, HIST)], sem_in)
            for j in range(NB)]

        for j in range(NB):
            fv = ff_v[pl.ds((g0 + j) * FFP, 16)]
            for f in range(N_FLOAT):
                v = fv[f]
                r = j * N_FLOAT + f
                for q in range(4):
                    dn_stage[r, pl.ds(q * 16, 16)] = (
                        ftab_v[pl.ds(f * D + q * 16, 16)] * v)

        ct.wait()
        for c in cseq:
            c.wait()

        for j in range(NB):
            g = g0 + j
            off = g * HISTP
            k0 = idx_seq_v[pl.ds(off, 16)]
            k1 = idx_seq_v[pl.ds(off + 16, 16)]
            k2 = idx_seq_v[pl.ds(off + 32, 16)]
            k3 = idx_seq_v[pl.ds(off + 48, 16)]
            nz = (plsc.all_reduce_population_count(k0 == 0)
                  + plsc.all_reduce_population_count(k1 == 0)
                  + plsc.all_reduce_population_count(k2 == 0)
                  + plsc.all_reduce_population_count((k3 == 0) & tail_mask))
            zf = nz.astype(jnp.float32)
            cnt = 50.0 - zf

            def body(h, accs):
                r = j * HIST + h
                return tuple(
                    accs[q] + seq_rows[r, pl.ds(q * 16, 16)] for q in range(4))

            zero = jnp.zeros((16,), jnp.float32)
            accs = lax.fori_loop(0, HIST, body, (zero, zero, zero, zero))
            for q in range(4):
                p = (accs[q] - zf * row0_v[0, pl.ds(q * 16, 16)]) / (cnt + 1e-8)
                pooled_v[j, pl.ds(q * 16, 16)] = jnp.where(cnt > 0.0, p, 0.0)

        orow = b0 * (N_TOK + 1)
        outs = []
        for j in range(NB):
            outs.append(pltpu.async_copy(
                tok_rows.at[pl.ds(j * N_TOK, N_TOK)],
                out_sp.at[pl.ds(orow + j * (N_TOK + 1), N_TOK)], sem_out))
            outs.append(pltpu.async_copy(
                pooled_v.at[pl.ds(j, 1)],
                out_sp.at[pl.ds(orow + j * (N_TOK + 1) + N_TOK, 1)], sem_out))
        outs.append(pltpu.async_copy(
            dn_stage, out_dn.at[pl.ds(b0 * N_FLOAT, NB * N_FLOAT)], sem_out))
        for o in outs:
            o.wait()
        return 0

    lax.fori_loop(0, NSTEPS, step, 0)


def kernel(token_fields, float_fields, token_seq_field, token_table,
           float_table, seq_table):
    offsets = (jnp.arange(N_TOK, dtype=jnp.int32) * TOK_DIM)[None, :]
    tok_idx = (token_fields.astype(jnp.int32) + offsets).reshape(-1)
    seq_idx = jnp.pad(token_seq_field.astype(jnp.int32),
                      ((0, 0), (0, HISTP - HIST)),
                      constant_values=1).reshape(-1)
    ffp = jnp.pad(float_fields, ((0, 0), (0, FFP - N_FLOAT))).reshape(-1)
    ftab = float_table.reshape(-1)

    out_sp, out_dn = _sc_embed(tok_idx, seq_idx, ffp, ftab,
                               token_table, seq_table)
    return (out_sp.reshape(B, N_TOK + 1, D), out_dn.reshape(B, N_FLOAT, D))
